# Initial kernel scaffold; baseline (speedup 1.0000x reference)
import jax
import jax.numpy as jnp
from jax.experimental import pallas as pl


def kernel(x, xyz, Wq, bq, Wk, bk, Wv, bv, Wo, bo, W1, b1, W2, b2):
    raise NotImplementedError("write your pallas kernel here")


# speedup vs baseline: 14.7637x; 14.7637x over previous
import functools

import jax
import jax.numpy as jnp
from jax import lax
from jax.experimental import pallas as pl
from jax.experimental.pallas import tpu as pltpu
from jax.experimental.pallas import tpu_sc as plsc

_B, _N, _DIM, _H, _K = 2, 4096, 256, 8, 16
_HD = _DIM // _H
_BN = _B * _N

_ROWS_PROJ = 512
_TR_KNN = 256
_TR_ATT = 128
_GW = 128


def _proj_body(x_ref, xyz_ref, wq_ref, bq_ref, wk_ref, bk_ref, wv_ref, bv_ref,
               w1_ref, q_ref, k_ref, v_ref, p_ref):
    xb = x_ref[...]
    f32 = jnp.float32
    q_ref[...] = jnp.dot(xb, wq_ref[...], preferred_element_type=f32) + bq_ref[...]
    k_ref[...] = jnp.dot(xb, wk_ref[...], preferred_element_type=f32) + bk_ref[...]
    v_ref[...] = jnp.dot(xb, wv_ref[...], preferred_element_type=f32) + bv_ref[...]
    xyzb = xyz_ref[...]
    p = (xyzb[:, 0:1] * w1_ref[0:1, :]
         + xyzb[:, 1:2] * w1_ref[1:2, :]
         + xyzb[:, 2:3] * w1_ref[2:3, :])
    p_ref[...] = jnp.concatenate([p, jnp.zeros((p.shape[0], 96), f32)], axis=1)


def _proj(x2, xyz2, Wq, bq, Wk, bk, Wv, bv, W1):
    grid = (_BN // _ROWS_PROJ,)
    row_spec = pl.BlockSpec((_ROWS_PROJ, _DIM), lambda i: (i, 0))
    full = lambda a: pl.BlockSpec(a.shape, lambda i: (0,) * a.ndim)
    return pl.pallas_call(
        _proj_body,
        grid=grid,
        in_specs=[
            row_spec,
            pl.BlockSpec((_ROWS_PROJ, 3), lambda i: (i, 0)),
            full(Wq), full(bq), full(Wk), full(bk), full(Wv), full(bv), full(W1),
        ],
        out_specs=[
            row_spec,
            row_spec,
            row_spec,
            pl.BlockSpec((_ROWS_PROJ, 128), lambda i: (i, 0)),
        ],
        out_shape=[
            jax.ShapeDtypeStruct((_BN, _DIM), jnp.float32),
            jax.ShapeDtypeStruct((_BN, _DIM), jnp.float32),
            jax.ShapeDtypeStruct((_BN, _DIM), jnp.float32),
            jax.ShapeDtypeStruct((_BN, 128), jnp.float32),
        ],
    )(x2, xyz2, Wq, bq, Wk, bk, Wv, bv, W1)


def _knn_body(xyzr_ref, xyzc_ref, idx_ref):
    b = pl.program_id(0)
    xr = xyzr_ref[0]
    xc = xyzc_ref[0]
    x0, x1, x2 = xr[:, 0:1], xr[:, 1:2], xr[:, 2:3]
    c0, c1, c2 = xc[0:1, :], xc[1:2, :], xc[2:3, :]
    g = jnp.dot(xr.astype(jnp.bfloat16), xc.astype(jnp.bfloat16),
                preferred_element_type=jnp.float32)
    sqr = x0 * x0 + x1 * x1 + x2 * x2
    sqc = c0 * c0 + c1 * c1 + c2 * c2
    work = sqr + sqc - 2.0 * g
    iota = lax.broadcasted_iota(jnp.int32, (_TR_KNN, _N), 1)
    big = jnp.int32(2 ** 30)
    inf = jnp.float32(jnp.inf)
    cols = []
    for _ in range(_K):
        m = jnp.min(work, axis=1, keepdims=True)
        cand = jnp.where(work <= m, iota, big)
        j = jnp.min(cand, axis=1, keepdims=True)
        cols.append(j)
        work = jnp.where(cand == j, inf, work)
    idxb = jnp.concatenate(cols, axis=1)
    idx_ref[0] = idxb + b * _N


def _knn(xyz, xyzT):
    grid = (_B, _N // _TR_KNN)
    return pl.pallas_call(
        _knn_body,
        grid=grid,
        in_specs=[
            pl.BlockSpec((1, _TR_KNN, 3), lambda b, i: (b, i, 0)),
            pl.BlockSpec((1, 3, _N), lambda b, i: (b, 0, 0)),
        ],
        out_specs=pl.BlockSpec((1, _TR_KNN, _K), lambda b, i: (b, i, 0)),
        out_shape=jax.ShapeDtypeStruct((_B, _N, _K), jnp.int32),
    )(xyz, xyzT)


def _sc_gather(k, v, p, idxf):
    n_idx = _BN * _K
    mesh = plsc.VectorSubcoreMesh(core_axis_name="core", subcore_axis_name="subcore")

    @functools.partial(
        pl.kernel,
        out_type=[
            jax.ShapeDtypeStruct((n_idx, _DIM), jnp.float32),
            jax.ShapeDtypeStruct((n_idx, _DIM), jnp.float32),
            jax.ShapeDtypeStruct((n_idx, 128), jnp.float32),
        ],
        mesh=mesh,
    )
    def gather_kernel(k_hbm, v_hbm, p_hbm, i_hbm, ok_hbm, ov_hbm, op_hbm):
        def pass_for(table_hbm, width):
            def body(i_vmem, o_vmem):
                pltpu.sync_copy(table_hbm.at[i_vmem.at[0]], o_vmem)
            return body

        for table, out_hbm, width in ((k_hbm, ok_hbm, _DIM),
                                      (v_hbm, ov_hbm, _DIM),
                                      (p_hbm, op_hbm, 128)):
            pltpu.emit_pipeline(
                pass_for(table, width),
                grid=(n_idx // _GW,),
                in_specs=[pl.BlockSpec((1, _GW), lambda i: (0, i))],
                out_specs=[pl.BlockSpec((_GW, width), lambda i: (i, 0))],
                core_axis_name=("core", "subcore"),
                dimension_semantics=(pltpu.PARALLEL,),
            )(i_hbm, out_hbm)

    return gather_kernel(k, v, p, idxf)


def _attn_body(q_ref, kn_ref, vn_ref, pn_ref, p_ref, b1_ref, w2_ref, b2_ref,
               wo_ref, bo_ref, y_ref):
    f32 = jnp.float32
    tr = _TR_ATT
    q = q_ref[...]
    kb = kn_ref[...]
    vb = vn_ref[...]
    pn = pn_ref[:, :32]
    pr = p_ref[:, :32] + b1_ref[...]

    pr_e = jnp.broadcast_to(pr[:, None, :], (tr, _K, 32)).reshape(tr * _K, 32)
    h = jnp.maximum(pr_e - pn, 0.0)
    bias = jnp.dot(h, w2_ref[...], preferred_element_type=f32) + b2_ref[...]

    q_e = jnp.broadcast_to(q[:, None, :], (tr, _K, _DIM)).reshape(tr * _K, _DIM)
    prod = q_e * kb
    iota_h = lax.broadcasted_iota(jnp.int32, (_DIM, _H), 0) // _HD
    iota_c = lax.broadcasted_iota(jnp.int32, (_DIM, _H), 1)
    seg = jnp.where(iota_h == iota_c, 1.0, 0.0).astype(f32)
    s = jnp.dot(prod, seg, preferred_element_type=f32)
    scores = s * (1.0 / (_HD ** 0.5)) + bias

    s3 = scores.reshape(tr, _K, _H)
    mx = jnp.max(s3, axis=1, keepdims=True)
    e = jnp.exp(s3 - mx)
    denom = jnp.sum(e, axis=1, keepdims=True)
    attn = (e / denom).reshape(tr * _K, _H)

    attn_e = jnp.dot(attn, seg.T, preferred_element_type=f32)
    o = (attn_e * vb).reshape(tr, _K, _DIM).sum(axis=1)
    y_ref[...] = jnp.dot(o, wo_ref[...], preferred_element_type=f32) + bo_ref[...]


def _attn(q, kn, vn, pn, p, b1, W2, b2, Wo, bo):
    grid = (_BN // _TR_ATT,)
    full = lambda a: pl.BlockSpec(a.shape, lambda i: (0,) * a.ndim)
    return pl.pallas_call(
        _attn_body,
        grid=grid,
        in_specs=[
            pl.BlockSpec((_TR_ATT, _DIM), lambda i: (i, 0)),
            pl.BlockSpec((_TR_ATT * _K, _DIM), lambda i: (i, 0)),
            pl.BlockSpec((_TR_ATT * _K, _DIM), lambda i: (i, 0)),
            pl.BlockSpec((_TR_ATT * _K, 128), lambda i: (i, 0)),
            pl.BlockSpec((_TR_ATT, 128), lambda i: (i, 0)),
            full(b1), full(W2), full(b2), full(Wo), full(bo),
        ],
        out_specs=pl.BlockSpec((_TR_ATT, _DIM), lambda i: (i, 0)),
        out_shape=jax.ShapeDtypeStruct((_BN, _DIM), jnp.float32),
    )(q, kn, vn, pn, p, b1, W2, b2, Wo, bo)


def kernel(x, xyz, Wq, bq, Wk, bk, Wv, bv, Wo, bo, W1, b1, W2, b2):
    x2 = x.reshape(_BN, _DIM)
    xyz2 = xyz.reshape(_BN, 3)
    q, k, v, p = _proj(x2, xyz2, Wq, bq.reshape(1, _DIM), Wk, bk.reshape(1, _DIM),
                       Wv, bv.reshape(1, _DIM), W1)
    xyzT = xyz.transpose(0, 2, 1)
    idx = _knn(xyz, xyzT)
    idxf = idx.reshape(1, _BN * _K)
    kn, vn, pn = _sc_gather(k, v, p, idxf)
    y = _attn(q, kn, vn, pn, p, b1.reshape(1, 32), W2, b2.reshape(1, _H),
              Wo, bo.reshape(1, _DIM))
    return y.reshape(_B, _N, _DIM)


# speedup vs baseline: 16.6235x; 1.1260x over previous
import functools

import jax
import jax.numpy as jnp
from jax import lax
from jax.experimental import pallas as pl
from jax.experimental.pallas import tpu as pltpu
from jax.experimental.pallas import tpu_sc as plsc

_B, _N, _DIM, _H, _K = 2, 4096, 256, 8, 16
_HD = _DIM // _H
_BN = _B * _N

_ROWS_PROJ = 512
_TR_KNN = 256
_TR_ATT = 128
_GW = 128


def _proj_body(x_ref, xyz_ref, wq_ref, bq_ref, wk_ref, bk_ref, wv_ref, bv_ref,
               w1_ref, q_ref, k_ref, v_ref, p_ref):
    xb = x_ref[...]
    f32 = jnp.float32
    q_ref[...] = jnp.dot(xb, wq_ref[...], preferred_element_type=f32) + bq_ref[...]
    k_ref[...] = jnp.dot(xb, wk_ref[...], preferred_element_type=f32) + bk_ref[...]
    v_ref[...] = jnp.dot(xb, wv_ref[...], preferred_element_type=f32) + bv_ref[...]
    xyzb = xyz_ref[...]
    p = (xyzb[:, 0:1] * w1_ref[0:1, :]
         + xyzb[:, 1:2] * w1_ref[1:2, :]
         + xyzb[:, 2:3] * w1_ref[2:3, :])
    p_ref[...] = jnp.concatenate([p, jnp.zeros((p.shape[0], 96), f32)], axis=1)


def _proj(x2, xyz2, Wq, bq, Wk, bk, Wv, bv, W1):
    grid = (_BN // _ROWS_PROJ,)
    row_spec = pl.BlockSpec((_ROWS_PROJ, _DIM), lambda i: (i, 0))
    full = lambda a: pl.BlockSpec(a.shape, lambda i: (0,) * a.ndim)
    return pl.pallas_call(
        _proj_body,
        grid=grid,
        compiler_params=pltpu.CompilerParams(dimension_semantics=("parallel",)),
        in_specs=[
            row_spec,
            pl.BlockSpec((_ROWS_PROJ, 3), lambda i: (i, 0)),
            full(Wq), full(bq), full(Wk), full(bk), full(Wv), full(bv), full(W1),
        ],
        out_specs=[
            row_spec,
            row_spec,
            row_spec,
            pl.BlockSpec((_ROWS_PROJ, 128), lambda i: (i, 0)),
        ],
        out_shape=[
            jax.ShapeDtypeStruct((_BN, _DIM), jnp.float32),
            jax.ShapeDtypeStruct((_BN, _DIM), jnp.float32),
            jax.ShapeDtypeStruct((_BN, _DIM), jnp.float32),
            jax.ShapeDtypeStruct((_BN, 128), jnp.float32),
        ],
    )(x2, xyz2, Wq, bq, Wk, bk, Wv, bv, W1)


def _knn_body(xyzr_ref, xyzc_ref, idx_ref):
    b = pl.program_id(0)
    xr = xyzr_ref[0]
    xc = xyzc_ref[0]
    x0, x1, x2 = xr[:, 0:1], xr[:, 1:2], xr[:, 2:3]
    c0, c1, c2 = xc[0:1, :], xc[1:2, :], xc[2:3, :]
    g = jnp.dot(xr.astype(jnp.bfloat16), xc.astype(jnp.bfloat16),
                preferred_element_type=jnp.float32)
    sqr = x0 * x0 + x1 * x1 + x2 * x2
    sqc = c0 * c0 + c1 * c1 + c2 * c2
    work = sqr + sqc - 2.0 * g
    iota_f = lax.broadcasted_iota(jnp.int32, (_TR_KNN, _N), 1).astype(jnp.float32)
    big_f = jnp.float32(1e9)
    inf = jnp.float32(jnp.inf)
    cols = []
    for _ in range(_K):
        m = jnp.min(work, axis=1, keepdims=True)
        cand = jnp.where(work <= m, iota_f, big_f)
        j = jnp.min(cand, axis=1, keepdims=True)
        cols.append(j)
        work = jnp.where(cand == j, inf, work)
    idxb = jnp.concatenate(cols, axis=1).astype(jnp.int32)
    idx_ref[0] = idxb + b * _N


def _knn(xyz, xyzT):
    grid = (_B, _N // _TR_KNN)
    return pl.pallas_call(
        _knn_body,
        grid=grid,
        compiler_params=pltpu.CompilerParams(
            dimension_semantics=("parallel", "parallel")),
        in_specs=[
            pl.BlockSpec((1, _TR_KNN, 3), lambda b, i: (b, i, 0)),
            pl.BlockSpec((1, 3, _N), lambda b, i: (b, 0, 0)),
        ],
        out_specs=pl.BlockSpec((1, _TR_KNN, _K), lambda b, i: (b, i, 0)),
        out_shape=jax.ShapeDtypeStruct((_B, _N, _K), jnp.int32),
    )(xyz, xyzT)


def _sc_gather(k, v, p, idxf):
    n_idx = _BN * _K
    mesh = plsc.VectorSubcoreMesh(core_axis_name="core", subcore_axis_name="subcore")

    @functools.partial(
        pl.kernel,
        out_type=[
            jax.ShapeDtypeStruct((n_idx, _DIM), jnp.float32),
            jax.ShapeDtypeStruct((n_idx, _DIM), jnp.float32),
            jax.ShapeDtypeStruct((n_idx, 128), jnp.float32),
        ],
        mesh=mesh,
    )
    def gather_kernel(k_hbm, v_hbm, p_hbm, i_hbm, ok_hbm, ov_hbm, op_hbm):
        def pass_for(table_hbm, width):
            def body(i_vmem, o_vmem):
                pltpu.sync_copy(table_hbm.at[i_vmem.at[0]], o_vmem)
            return body

        for table, out_hbm, width in ((k_hbm, ok_hbm, _DIM),
                                      (v_hbm, ov_hbm, _DIM),
                                      (p_hbm, op_hbm, 128)):
            pltpu.emit_pipeline(
                pass_for(table, width),
                grid=(n_idx // _GW,),
                in_specs=[pl.BlockSpec((1, _GW), lambda i: (0, i))],
                out_specs=[pl.BlockSpec((_GW, width), lambda i: (i, 0))],
                core_axis_name=("core", "subcore"),
                dimension_semantics=(pltpu.PARALLEL,),
            )(i_hbm, out_hbm)

    return gather_kernel(k, v, p, idxf)


def _attn_body(q_ref, kn_ref, vn_ref, pn_ref, p_ref, b1_ref, w2_ref, b2_ref,
               wo_ref, bo_ref, y_ref):
    f32 = jnp.float32
    tr = _TR_ATT
    q = q_ref[...]
    kb = kn_ref[...]
    vb = vn_ref[...]
    pn = pn_ref[:, :32]
    pr = p_ref[:, :32] + b1_ref[...]

    pr_e = jnp.broadcast_to(pr[:, None, :], (tr, _K, 32)).reshape(tr * _K, 32)
    h = jnp.maximum(pr_e - pn, 0.0)
    bias = jnp.dot(h, w2_ref[...], preferred_element_type=f32) + b2_ref[...]

    q_e = jnp.broadcast_to(q[:, None, :], (tr, _K, _DIM)).reshape(tr * _K, _DIM)
    prod = q_e * kb
    iota_h = lax.broadcasted_iota(jnp.int32, (_DIM, _H), 0) // _HD
    iota_c = lax.broadcasted_iota(jnp.int32, (_DIM, _H), 1)
    seg = jnp.where(iota_h == iota_c, 1.0, 0.0).astype(f32)
    s = jnp.dot(prod, seg, preferred_element_type=f32)
    scores = s * (1.0 / (_HD ** 0.5)) + bias

    s3 = scores.reshape(tr, _K, _H)
    mx = jnp.max(s3, axis=1, keepdims=True)
    e = jnp.exp(s3 - mx)
    denom = jnp.sum(e, axis=1, keepdims=True)
    attn = (e / denom).reshape(tr * _K, _H)

    attn_e = jnp.dot(attn, seg.T, preferred_element_type=f32)
    o = (attn_e * vb).reshape(tr, _K, _DIM).sum(axis=1)
    y_ref[...] = jnp.dot(o, wo_ref[...], preferred_element_type=f32) + bo_ref[...]


def _attn(q, kn, vn, pn, p, b1, W2, b2, Wo, bo):
    grid = (_BN // _TR_ATT,)
    full = lambda a: pl.BlockSpec(a.shape, lambda i: (0,) * a.ndim)
    return pl.pallas_call(
        _attn_body,
        grid=grid,
        compiler_params=pltpu.CompilerParams(dimension_semantics=("parallel",)),
        in_specs=[
            pl.BlockSpec((_TR_ATT, _DIM), lambda i: (i, 0)),
            pl.BlockSpec((_TR_ATT * _K, _DIM), lambda i: (i, 0)),
            pl.BlockSpec((_TR_ATT * _K, _DIM), lambda i: (i, 0)),
            pl.BlockSpec((_TR_ATT * _K, 128), lambda i: (i, 0)),
            pl.BlockSpec((_TR_ATT, 128), lambda i: (i, 0)),
            full(b1), full(W2), full(b2), full(Wo), full(bo),
        ],
        out_specs=pl.BlockSpec((_TR_ATT, _DIM), lambda i: (i, 0)),
        out_shape=jax.ShapeDtypeStruct((_BN, _DIM), jnp.float32),
    )(q, kn, vn, pn, p, b1, W2, b2, Wo, bo)


def kernel(x, xyz, Wq, bq, Wk, bk, Wv, bv, Wo, bo, W1, b1, W2, b2):
    x2 = x.reshape(_BN, _DIM)
    xyz2 = xyz.reshape(_BN, 3)
    q, k, v, p = _proj(x2, xyz2, Wq, bq.reshape(1, _DIM), Wk, bk.reshape(1, _DIM),
                       Wv, bv.reshape(1, _DIM), W1)
    xyzT = xyz.transpose(0, 2, 1)
    idx = _knn(xyz, xyzT)
    idxf = idx.reshape(1, _BN * _K)
    kn, vn, pn = _sc_gather(k, v, p, idxf)
    y = _attn(q, kn, vn, pn, p, b1.reshape(1, 32), W2, b2.reshape(1, _H),
              Wo, bo.reshape(1, _DIM))
    return y.reshape(_B, _N, _DIM)


# speedup vs baseline: 19.6455x; 1.1818x over previous
import functools

import jax
import jax.numpy as jnp
from jax import lax
from jax.experimental import pallas as pl
from jax.experimental.pallas import tpu as pltpu
from jax.experimental.pallas import tpu_sc as plsc

_B, _N, _DIM, _H, _K = 2, 4096, 256, 8, 16
_HD = _DIM // _H
_BN = _B * _N

_ROWS_PROJ = 512
_TR_KNN = 256
_TR_ATT = 128
_GW = 128


def _proj_body(x_ref, xyz_ref, wq_ref, bq_ref, wk_ref, bk_ref, wv_ref, bv_ref,
               w1_ref, q_ref, k_ref, v_ref, p_ref):
    xb = x_ref[...]
    f32 = jnp.float32
    q_ref[...] = jnp.dot(xb, wq_ref[...], preferred_element_type=f32) + bq_ref[...]
    k_ref[...] = jnp.dot(xb, wk_ref[...], preferred_element_type=f32) + bk_ref[...]
    v_ref[...] = jnp.dot(xb, wv_ref[...], preferred_element_type=f32) + bv_ref[...]
    xyzb = xyz_ref[...]
    p = (xyzb[:, 0:1] * w1_ref[0:1, :]
         + xyzb[:, 1:2] * w1_ref[1:2, :]
         + xyzb[:, 2:3] * w1_ref[2:3, :])
    p_ref[...] = jnp.concatenate([p, jnp.zeros((p.shape[0], 96), f32)], axis=1)


def _proj(x2, xyz2, Wq, bq, Wk, bk, Wv, bv, W1):
    grid = (_BN // _ROWS_PROJ,)
    row_spec = pl.BlockSpec((_ROWS_PROJ, _DIM), lambda i: (i, 0))
    full = lambda a: pl.BlockSpec(a.shape, lambda i: (0,) * a.ndim)
    return pl.pallas_call(
        _proj_body,
        grid=grid,
        compiler_params=pltpu.CompilerParams(dimension_semantics=("parallel",)),
        in_specs=[
            row_spec,
            pl.BlockSpec((_ROWS_PROJ, 3), lambda i: (i, 0)),
            full(Wq), full(bq), full(Wk), full(bk), full(Wv), full(bv), full(W1),
        ],
        out_specs=[
            row_spec,
            row_spec,
            row_spec,
            pl.BlockSpec((_ROWS_PROJ, 128), lambda i: (i, 0)),
        ],
        out_shape=[
            jax.ShapeDtypeStruct((_BN, _DIM), jnp.float32),
            jax.ShapeDtypeStruct((_BN, _DIM), jnp.float32),
            jax.ShapeDtypeStruct((_BN, _DIM), jnp.float32),
            jax.ShapeDtypeStruct((_BN, 128), jnp.float32),
        ],
    )(x2, xyz2, Wq, bq, Wk, bk, Wv, bv, W1)


def _knn_body(b, xyzr_ref, xyzc_ref, idx_ref):
    xr = xyzr_ref[0]
    xc = xyzc_ref[0]
    x0, x1, x2 = xr[:, 0:1], xr[:, 1:2], xr[:, 2:3]
    c0, c1, c2 = xc[0:1, :], xc[1:2, :], xc[2:3, :]
    g = jnp.dot(xr.astype(jnp.bfloat16), xc.astype(jnp.bfloat16),
                preferred_element_type=jnp.float32)
    sqr = x0 * x0 + x1 * x1 + x2 * x2
    sqc = c0 * c0 + c1 * c1 + c2 * c2
    work = sqr + sqc - 2.0 * g
    iota_f = lax.broadcasted_iota(jnp.int32, (_TR_KNN, _N), 1).astype(jnp.float32)
    big_f = jnp.float32(1e9)
    inf = jnp.float32(jnp.inf)
    cols = []
    for _ in range(_K):
        m = jnp.min(work, axis=1, keepdims=True)
        cand = jnp.where(work <= m, iota_f, big_f)
        j = jnp.min(cand, axis=1, keepdims=True)
        cols.append(j)
        work = jnp.where(cand == j, inf, work)
    idxb = jnp.concatenate(cols, axis=1).astype(jnp.int32)
    idx_ref[...] = idxb + b * _N


def _knn(xyz, xyzT, b):
    grid = (_N // _TR_KNN,)
    return pl.pallas_call(
        functools.partial(_knn_body, b),
        grid=grid,
        compiler_params=pltpu.CompilerParams(dimension_semantics=("parallel",)),
        in_specs=[
            pl.BlockSpec((1, _TR_KNN, 3), lambda i: (b, i, 0)),
            pl.BlockSpec((1, 3, _N), lambda i: (b, 0, 0)),
        ],
        out_specs=pl.BlockSpec((_TR_KNN, _K), lambda i: (i, 0)),
        out_shape=jax.ShapeDtypeStruct((_N, _K), jnp.int32),
    )(xyz, xyzT)


def _sc_gather(k, v, p, idxf):
    n_idx = idxf.shape[1]
    mesh = plsc.VectorSubcoreMesh(core_axis_name="core", subcore_axis_name="subcore")

    @functools.partial(
        pl.kernel,
        out_type=[
            jax.ShapeDtypeStruct((n_idx, _DIM), jnp.float32),
            jax.ShapeDtypeStruct((n_idx, _DIM), jnp.float32),
            jax.ShapeDtypeStruct((n_idx, 128), jnp.float32),
        ],
        mesh=mesh,
    )
    def gather_kernel(k_hbm, v_hbm, p_hbm, i_hbm, ok_hbm, ov_hbm, op_hbm):
        def pass_for(table_hbm):
            def body(i_vmem, o_vmem):
                pltpu.sync_copy(table_hbm.at[i_vmem.at[0]], o_vmem)
            return body

        for table, out_hbm, width in ((k_hbm, ok_hbm, _DIM),
                                      (v_hbm, ov_hbm, _DIM),
                                      (p_hbm, op_hbm, 128)):
            pltpu.emit_pipeline(
                pass_for(table),
                grid=(n_idx // _GW,),
                in_specs=[pl.BlockSpec((1, _GW), lambda i: (0, i))],
                out_specs=[pl.BlockSpec((_GW, width), lambda i: (i, 0))],
                core_axis_name=("core", "subcore"),
                dimension_semantics=(pltpu.PARALLEL,),
            )(i_hbm, out_hbm)

    return gather_kernel(k, v, p, idxf)


def _attn_body(q_ref, kn_ref, vn_ref, pn_ref, p_ref, b1_ref, w2_ref, b2_ref,
               wo_ref, bo_ref, y_ref):
    f32 = jnp.float32
    tr = _TR_ATT
    q = q_ref[...]
    kb = kn_ref[...]
    vb = vn_ref[...]
    pn = pn_ref[:, :32]
    pr = p_ref[:, :32] + b1_ref[...]

    pr_e = jnp.broadcast_to(pr[:, None, :], (tr, _K, 32)).reshape(tr * _K, 32)
    h = jnp.maximum(pr_e - pn, 0.0)
    bias = jnp.dot(h, w2_ref[...], preferred_element_type=f32) + b2_ref[...]

    q_e = jnp.broadcast_to(q[:, None, :], (tr, _K, _DIM)).reshape(tr * _K, _DIM)
    prod = q_e * kb
    iota_h = lax.broadcasted_iota(jnp.int32, (_DIM, _H), 0) // _HD
    iota_c = lax.broadcasted_iota(jnp.int32, (_DIM, _H), 1)
    seg = jnp.where(iota_h == iota_c, 1.0, 0.0).astype(f32)
    s = jnp.dot(prod, seg, preferred_element_type=f32)
    scores = s * (1.0 / (_HD ** 0.5)) + bias

    s3 = scores.reshape(tr, _K, _H)
    mx = jnp.max(s3, axis=1, keepdims=True)
    e = jnp.exp(s3 - mx)
    denom = jnp.sum(e, axis=1, keepdims=True)
    attn = (e / denom).reshape(tr * _K, _H)

    attn_e = jnp.dot(attn, seg.T, preferred_element_type=f32)
    o = (attn_e * vb).reshape(tr, _K, _DIM).sum(axis=1)
    y_ref[...] = jnp.dot(o, wo_ref[...], preferred_element_type=f32) + bo_ref[...]


def _attn(q, kn, vn, pn, p, b1, W2, b2, Wo, bo, row0, n_rows):
    grid = (n_rows // _TR_ATT,)
    r0 = row0 // _TR_ATT
    full = lambda a: pl.BlockSpec(a.shape, lambda i: (0,) * a.ndim)
    return pl.pallas_call(
        _attn_body,
        grid=grid,
        compiler_params=pltpu.CompilerParams(dimension_semantics=("parallel",)),
        in_specs=[
            pl.BlockSpec((_TR_ATT, _DIM), lambda i: (r0 + i, 0)),
            pl.BlockSpec((_TR_ATT * _K, _DIM), lambda i: (i, 0)),
            pl.BlockSpec((_TR_ATT * _K, _DIM), lambda i: (i, 0)),
            pl.BlockSpec((_TR_ATT * _K, 128), lambda i: (i, 0)),
            pl.BlockSpec((_TR_ATT, 128), lambda i: (r0 + i, 0)),
            full(b1), full(W2), full(b2), full(Wo), full(bo),
        ],
        out_specs=pl.BlockSpec((_TR_ATT, _DIM), lambda i: (i, 0)),
        out_shape=jax.ShapeDtypeStruct((n_rows, _DIM), jnp.float32),
    )(q, kn, vn, pn, p, b1, W2, b2, Wo, bo)


def kernel(x, xyz, Wq, bq, Wk, bk, Wv, bv, Wo, bo, W1, b1, W2, b2):
    x2 = x.reshape(_BN, _DIM)
    xyz2 = xyz.reshape(_BN, 3)
    q, k, v, p = _proj(x2, xyz2, Wq, bq.reshape(1, _DIM), Wk, bk.reshape(1, _DIM),
                       Wv, bv.reshape(1, _DIM), W1)
    xyzT = xyz.transpose(0, 2, 1)
    idxs = [_knn(xyz, xyzT, b).reshape(1, _N * _K) for b in range(_B)]
    gathered = [_sc_gather(k, v, p, idxs[b]) for b in range(_B)]
    ys = [_attn(q, *gathered[b], p, b1.reshape(1, 32), W2, b2.reshape(1, _H),
                Wo, bo.reshape(1, _DIM), b * _N, _N)
          for b in range(_B)]
    return jnp.concatenate(ys, axis=0).reshape(_B, _N, _DIM)
